# baseline (speedup 1.0000x reference)
import jax
import jax.numpy as jnp
from jax.experimental import pallas as pl
from jax.experimental.pallas import tpu as pltpu

_N = 4096
_DP = 128
_D = 64
_RB = 1024
_NRB = _N // _RB
_STEPS = 6 * _NRB


def _adj_block_index(n, p0, badj, bsrc, bdst):
    j = jnp.mod(n - _NRB, 4)
    ai = jnp.where(n < _NRB, p0[0],
                   jnp.where(n < 5 * _NRB, badj[j], p0[1]))
    rr = jnp.where(n < _NRB, n,
                   jnp.where(n < 5 * _NRB, (n - _NRB) // 4, n - 5 * _NRB))
    return ai, rr, 0


def _cell_kernel(p0_ref, badj_ref, bsrc_ref, bdst_ref, x_ref, w_ref, b_ref,
                 g_ref, bt_ref, adj_ref, o_ref, states_ref):
    n = pl.program_id(0)
    a = adj_ref[0].astype(jnp.bfloat16)

    @pl.when(n == 0)
    def _():
        h = jax.lax.dot_general(x_ref[...], w_ref[...],
                                (((1,), (1,)), ((), ())),
                                preferred_element_type=jnp.float32)
        states_ref[0] = h + b_ref[0][None, :]

    @pl.when(n < _NRB)
    def _():
        row = pl.ds(n * _RB, _RB)
        states_ref[1, row] = jnp.dot(
            a, states_ref[0].astype(jnp.bfloat16),
            preferred_element_type=jnp.float32)

    @pl.when(jnp.logical_and(n >= _NRB, n < 5 * _NRB))
    def _():
        j = jnp.mod(n - _NRB, 4)
        rbB = (n - _NRB) // 4
        row = pl.ds(rbB * _RB, _RB)

        @pl.when(j == 0)
        def _():
            states_ref[2, row] = jnp.zeros((_RB, _D), jnp.float32)
            states_ref[3, row] = jnp.zeros((_RB, _D), jnp.float32)

        src = bsrc_ref[j]
        dst = bdst_ref[j]
        rhs = states_ref[src].astype(jnp.bfloat16)
        contrib = jnp.dot(a, rhs, preferred_element_type=jnp.float32)
        states_ref[dst, row] += contrib

    @pl.when(n >= 5 * _NRB)
    def _():
        row = pl.ds((n - 5 * _NRB) * _RB, _RB)
        s = states_ref[3, row] + jnp.dot(
            a, states_ref[2].astype(jnp.bfloat16),
            preferred_element_type=jnp.float32)
        mu = jnp.mean(s, axis=-1, keepdims=True)
        var = jnp.mean((s - mu) ** 2, axis=-1, keepdims=True)
        ln = (s - mu) * jax.lax.rsqrt(var + 1e-5) * g_ref[0][None, :] \
            + bt_ref[0][None, :]
        o_ref[...] = 0.5 * ln * (1.0 + jax.lax.erf(ln * 0.7071067811865476))


def kernel(x, adjs, idxes_seq, idxes_res, W, b, gamma, beta):
    iseq = idxes_seq.astype(jnp.int32)
    ires = idxes_res.astype(jnp.int32)
    b_adj = jnp.stack([iseq[1], ires[0], ires[1], ires[2]])
    b_src = jnp.array([1, 0, 0, 1], jnp.int32)
    b_dst = jnp.array([2, 2, 3, 3], jnp.int32)
    perm = jnp.argsort(b_adj)
    b_adj = b_adj[perm]
    b_src = b_src[perm]
    b_dst = b_dst[perm]
    p0 = jnp.stack([iseq[0], iseq[2]])

    grid_spec = pltpu.PrefetchScalarGridSpec(
        num_scalar_prefetch=4,
        grid=(_STEPS,),
        in_specs=[
            pl.BlockSpec((_N, _DP), lambda n, *s: (0, 0)),
            pl.BlockSpec((_D, _DP), lambda n, *s: (0, 0)),
            pl.BlockSpec((1, _D), lambda n, *s: (0, 0)),
            pl.BlockSpec((1, _D), lambda n, *s: (0, 0)),
            pl.BlockSpec((1, _D), lambda n, *s: (0, 0)),
            pl.BlockSpec((1, _RB, _N), _adj_block_index),
        ],
        out_specs=pl.BlockSpec(
            (_RB, _D),
            lambda n, *s: (jnp.where(n >= 5 * _NRB, n - 5 * _NRB, 0), 0)),
        scratch_shapes=[pltpu.VMEM((4, _N, _D), jnp.float32)],
    )
    return pl.pallas_call(
        _cell_kernel,
        grid_spec=grid_spec,
        out_shape=jax.ShapeDtypeStruct((_N, _D), jnp.float32),
        compiler_params=pltpu.CompilerParams(
            vmem_limit_bytes=100 * 1024 * 1024),
    )(p0, b_adj, b_src, b_dst, x, W, b.reshape(1, _D), gamma.reshape(1, _D),
      beta.reshape(1, _D), adjs)


# speedup vs baseline: 1.1503x; 1.1503x over previous
import jax
import jax.numpy as jnp
from jax.experimental import pallas as pl
from jax.experimental.pallas import tpu as pltpu

_N = 4096
_DP = 128
_D = 64
_RB = 512
_NRB = _N // _RB
_STEPS = 6 * _NRB
_NBUF = 4


def _cell_kernel(ai_ref, rr_ref, fetch_ref, slot_ref, bsrc_ref, bdst_ref,
                 x_ref, w_ref, b_ref, g_ref, bt_ref, adj_ref, o_ref,
                 states_ref, buf_ref, sem_ref):
    n = pl.program_id(0)

    def _copy(m):
        return pltpu.make_async_copy(
            adj_ref.at[ai_ref[m], pl.ds(rr_ref[m] * _RB, _RB), :],
            buf_ref.at[slot_ref[m]],
            sem_ref.at[slot_ref[m]])

    @pl.when(n == 0)
    def _():
        for k in range(1, _NBUF - 1):
            @pl.when(fetch_ref[k] == 1)
            def _(k=k):
                _copy(k).start()

    @pl.when(n == 0)
    def _():
        _copy(0).start()
        h = jax.lax.dot_general(x_ref[...], w_ref[...],
                                (((1,), (1,)), ((), ())),
                                preferred_element_type=jnp.float32)
        states_ref[0] = h + b_ref[0][None, :]

    m = jnp.minimum(n + _NBUF - 1, _STEPS - 1)

    @pl.when(jnp.logical_and(n + _NBUF - 1 < _STEPS, fetch_ref[m] == 1))
    def _():
        _copy(m).start()

    @pl.when(fetch_ref[n] == 1)
    def _():
        _copy(n).wait()

    a = buf_ref[slot_ref[n]].astype(jnp.bfloat16)

    @pl.when(n < _NRB)
    def _():
        row = pl.ds(n * _RB, _RB)
        states_ref[1, row] = jnp.dot(
            a, states_ref[0].astype(jnp.bfloat16),
            preferred_element_type=jnp.float32)

    @pl.when(jnp.logical_and(n >= _NRB, n < 5 * _NRB))
    def _():
        j = jnp.mod(n - _NRB, 4)
        rbB = (n - _NRB) // 4
        row = pl.ds(rbB * _RB, _RB)

        @pl.when(j == 0)
        def _():
            states_ref[2, row] = jnp.zeros((_RB, _D), jnp.float32)
            states_ref[3, row] = jnp.zeros((_RB, _D), jnp.float32)

        src = bsrc_ref[j]
        dst = bdst_ref[j]
        rhs = states_ref[src].astype(jnp.bfloat16)
        contrib = jnp.dot(a, rhs, preferred_element_type=jnp.float32)
        states_ref[dst, row] += contrib

    @pl.when(n >= 5 * _NRB)
    def _():
        row = pl.ds((n - 5 * _NRB) * _RB, _RB)
        s = states_ref[3, row] + jnp.dot(
            a, states_ref[2].astype(jnp.bfloat16),
            preferred_element_type=jnp.float32)
        mu = jnp.mean(s, axis=-1, keepdims=True)
        var = jnp.mean((s - mu) ** 2, axis=-1, keepdims=True)
        ln = (s - mu) * jax.lax.rsqrt(var + 1e-5) * g_ref[0][None, :] \
            + bt_ref[0][None, :]
        o_ref[...] = 0.5 * ln * (1.0 + jax.lax.erf(ln * 0.7071067811865476))


def kernel(x, adjs, idxes_seq, idxes_res, W, b, gamma, beta):
    iseq = idxes_seq.astype(jnp.int32)
    ires = idxes_res.astype(jnp.int32)
    b_adj = jnp.stack([iseq[1], ires[0], ires[1], ires[2]])
    b_src = jnp.array([1, 0, 0, 1], jnp.int32)
    b_dst = jnp.array([2, 2, 3, 3], jnp.int32)
    perm = jnp.argsort(b_adj)
    b_adj = b_adj[perm]
    b_src = b_src[perm]
    b_dst = b_dst[perm]

    blk = jnp.arange(_NRB, dtype=jnp.int32)
    ai_all = jnp.concatenate([
        jnp.full((_NRB,), iseq[0], jnp.int32),
        jnp.tile(b_adj, _NRB),
        jnp.full((_NRB,), iseq[2], jnp.int32)])
    rr_all = jnp.concatenate([blk, jnp.repeat(blk, 4), blk])
    key = ai_all * _NRB + rr_all
    fetch = jnp.concatenate([
        jnp.ones((1,), jnp.int32), (key[1:] != key[:-1]).astype(jnp.int32)])
    slot = jnp.mod(jnp.cumsum(fetch) - 1, _NBUF).astype(jnp.int32)

    grid_spec = pltpu.PrefetchScalarGridSpec(
        num_scalar_prefetch=6,
        grid=(_STEPS,),
        in_specs=[
            pl.BlockSpec((_N, _DP), lambda n, *s: (0, 0)),
            pl.BlockSpec((_D, _DP), lambda n, *s: (0, 0)),
            pl.BlockSpec((1, _D), lambda n, *s: (0, 0)),
            pl.BlockSpec((1, _D), lambda n, *s: (0, 0)),
            pl.BlockSpec((1, _D), lambda n, *s: (0, 0)),
            pl.BlockSpec(memory_space=pltpu.MemorySpace.HBM),
        ],
        out_specs=pl.BlockSpec(
            (_RB, _D),
            lambda n, *s: (jnp.where(n >= 5 * _NRB, n - 5 * _NRB, 0), 0)),
        scratch_shapes=[
            pltpu.VMEM((4, _N, _D), jnp.float32),
            pltpu.VMEM((_NBUF, _RB, _N), jnp.float32),
            pltpu.SemaphoreType.DMA((_NBUF,)),
        ],
    )
    return pl.pallas_call(
        _cell_kernel,
        grid_spec=grid_spec,
        out_shape=jax.ShapeDtypeStruct((_N, _D), jnp.float32),
        compiler_params=pltpu.CompilerParams(
            vmem_limit_bytes=100 * 1024 * 1024),
    )(ai_all, rr_all, fetch, slot, b_src, b_dst, x, W, b.reshape(1, _D),
      gamma.reshape(1, _D), beta.reshape(1, _D), adjs)
